# baseline (speedup 1.0000x reference)
import functools

import jax
import jax.numpy as jnp
from jax.experimental import pallas as pl
from jax.experimental.pallas import tpu as pltpu

_CHOICE_TEMPERATURE = 4.5
_INT_MIN = -2147483648


def _body(temp_ref, ml_ref, la_ref, lb_ref, vidx_ref, gumbel_ref, mask_ref,
          zpred_ref, maskbc_ref, conf_ref, *, s_blk, s, v, nb, nj):
    bi = pl.program_id(0)
    j = pl.program_id(1)
    h = v // 2
    xa = la_ref[0]
    xb = lb_ref[0]
    m = jnp.maximum(jnp.max(xa, axis=-1), jnp.max(xb, axis=-1))
    ca = jnp.where(xa == m[:, None], vidx_ref[:, :h], v)
    cb = jnp.where(xb == m[:, None], vidx_ref[:, h:], v)
    amax = jnp.minimum(jnp.min(ca, axis=-1), jnp.min(cb, axis=-1))
    se = jnp.sum(jnp.exp(xa - m[:, None]), axis=-1) + \
        jnp.sum(jnp.exp(xb - m[:, None]), axis=-1)
    pmax = 1.0 / se
    temp = temp_ref[0]
    g = gumbel_ref[0, 0]
    mk = mask_ref[0, 0]
    conf = jnp.where(mk != 0, pmax + temp * g, jnp.inf)
    zpred_ref[0, 0, :] = amax
    conf_ref[0, pl.ds(bi * s + j * s_blk, s_blk)] = conf

    @pl.when((bi == nb - 1) & (j == nj - 1))
    def _rank():
        c = conf_ref[0, :].reshape(nb, s)
        cbits = jax.lax.bitcast_convert_type(c, jnp.int32)
        key = jnp.where(cbits < 0, cbits ^ 0x7FFFFFFF, cbits)
        k = ml_ref[0]
        imin = jnp.int32(_INT_MIN)

        def bit1(i, res_u):
            cand_u = res_u | (jnp.int32(1) << (31 - i))
            cnt = jnp.sum((key < (cand_u ^ imin)).astype(jnp.int32),
                          axis=1, keepdims=True)
            return jnp.where(cnt < k, cand_u, res_u)

        res_u = jax.lax.fori_loop(0, 32, bit1, jnp.zeros((nb, 1), jnp.int32))
        t_s = res_u ^ imin
        lt = key < t_s
        eq = key == t_s
        jrem = k - jnp.sum(lt.astype(jnp.int32), axis=1, keepdims=True)
        idx = vidx_ref[:, :s]

        def bit2(i, res2):
            cand2 = res2 | (jnp.int32(1) << (9 - i))
            cnt = jnp.sum((eq & (idx < cand2)).astype(jnp.int32),
                          axis=1, keepdims=True)
            return jnp.where(cnt < jrem, cand2, res2)

        t_idx = jax.lax.fori_loop(0, 10, bit2, jnp.zeros((nb, 1), jnp.int32))
        maskbc_ref[:, :] = (lt | (eq & (idx <= t_idx))).astype(jnp.int32)


def kernel(logits, ratio, gumbel, z_indices, mask, mask_num):
    del z_indices
    b, s, v = logits.shape
    s_blk = 128
    nj = s // s_blk
    h = v // 2

    r = ratio[0]
    mask_ratio = jnp.cos(r * jnp.pi / 2.0)
    mask_len = jnp.maximum(jnp.ceil(mask_num * mask_ratio), 1.0).astype(jnp.int32)
    temperature = (_CHOICE_TEMPERATURE * (1.0 - mask_ratio)).astype(jnp.float32)

    vidx = jnp.arange(v, dtype=jnp.int32).reshape(1, v)
    gumbel3 = gumbel.reshape(b * nj, 1, s_blk)
    mask3 = mask.astype(jnp.int32).reshape(b * nj, 1, s_blk)

    zpred, maskbc = pl.pallas_call(
        functools.partial(_body, s_blk=s_blk, s=s, v=v, nb=b, nj=nj),
        grid=(b, nj),
        in_specs=[
            pl.BlockSpec(memory_space=pltpu.SMEM),
            pl.BlockSpec(memory_space=pltpu.SMEM),
            pl.BlockSpec((1, s_blk, h), lambda bi, ji: (bi, ji, 0)),
            pl.BlockSpec((1, s_blk, h), lambda bi, ji: (bi, ji, 1)),
            pl.BlockSpec((1, v), lambda bi, ji: (0, 0)),
            pl.BlockSpec((1, 1, s_blk), lambda bi, ji: (bi * nj + ji, 0, 0)),
            pl.BlockSpec((1, 1, s_blk), lambda bi, ji: (bi * nj + ji, 0, 0)),
        ],
        out_specs=[
            pl.BlockSpec((1, 1, s_blk), lambda bi, ji: (bi * nj + ji, 0, 0)),
            pl.BlockSpec((b, s), lambda bi, ji: (0, 0)),
        ],
        out_shape=[
            jax.ShapeDtypeStruct((b * nj, 1, s_blk), jnp.int32),
            jax.ShapeDtypeStruct((b, s), jnp.int32),
        ],
        scratch_shapes=[pltpu.VMEM((1, b * s), jnp.float32)],
    )(temperature.reshape(1), mask_len.reshape(1), logits, logits, vidx,
      gumbel3, mask3)

    return zpred.reshape(b, s), maskbc.astype(jnp.bool_)


# speedup vs baseline: 1.2114x; 1.2114x over previous
import functools

import jax
import jax.numpy as jnp
from jax.experimental import pallas as pl
from jax.experimental.pallas import tpu as pltpu

_CHOICE_TEMPERATURE = 4.5
_INT_MIN = -2147483648


def _body(temp_ref, ml_ref, logits_ref, vidx_ref, gumbel_ref, mask_ref,
          zpred_ref, maskbc_ref, conf_ref, *, s_blk, s, v, nb, nj):
    bi = pl.program_id(0)
    j = pl.program_id(1)
    x = logits_ref[0]
    m = jnp.max(x, axis=-1)
    cand = jnp.where(x == m[:, None], vidx_ref[:], v)
    amax = jnp.min(cand, axis=-1)
    se = jnp.sum(jnp.exp(x - m[:, None]), axis=-1)
    pmax = 1.0 / se
    temp = temp_ref[0]
    g = gumbel_ref[0, 0]
    mk = mask_ref[0, 0]
    conf = jnp.where(mk != 0, pmax + temp * g, jnp.inf)
    zpred_ref[0, 0, :] = amax
    conf_ref[0, pl.ds(bi * s + j * s_blk, s_blk)] = conf

    @pl.when((bi == nb - 1) & (j == nj - 1))
    def _rank():
        c = conf_ref[0, :].reshape(nb, s)
        cb = jax.lax.bitcast_convert_type(c, jnp.int32)
        key = jnp.where(cb < 0, cb ^ 0x7FFFFFFF, cb)
        k = ml_ref[0]
        imin = jnp.int32(_INT_MIN)

        def bit1(i, res_u):
            cand_u = res_u | (jnp.int32(1) << (31 - i))
            cnt = jnp.sum((key < (cand_u ^ imin)).astype(jnp.int32),
                          axis=1, keepdims=True)
            return jnp.where(cnt < k, cand_u, res_u)

        res_u = jax.lax.fori_loop(0, 32, bit1,
                                  jnp.zeros((nb, 1), jnp.int32))
        t_s = res_u ^ imin
        lt = key < t_s
        eq = key == t_s
        jrem = k - jnp.sum(lt.astype(jnp.int32), axis=1, keepdims=True)
        idx = vidx_ref[:, :s]

        def bit2(i, res2):
            cand2 = res2 | (jnp.int32(1) << (9 - i))
            cnt = jnp.sum((eq & (idx < cand2)).astype(jnp.int32),
                          axis=1, keepdims=True)
            return jnp.where(cnt < jrem, cand2, res2)

        t_idx = jax.lax.fori_loop(0, 10, bit2,
                                  jnp.zeros((nb, 1), jnp.int32))
        maskbc_ref[:, :] = (lt | (eq & (idx <= t_idx))).astype(jnp.int32)


def kernel(logits, ratio, gumbel, z_indices, mask, mask_num):
    del z_indices
    b, s, v = logits.shape
    s_blk = 256
    nj = s // s_blk

    r = ratio[0]
    mask_ratio = jnp.cos(r * jnp.pi / 2.0)
    mask_len = jnp.maximum(jnp.ceil(mask_num * mask_ratio), 1.0).astype(jnp.int32)
    temperature = (_CHOICE_TEMPERATURE * (1.0 - mask_ratio)).astype(jnp.float32)

    vidx = jnp.arange(v, dtype=jnp.int32).reshape(1, v)
    gumbel3 = gumbel.reshape(b * nj, 1, s_blk)
    mask3 = mask.astype(jnp.int32).reshape(b * nj, 1, s_blk)

    zpred, maskbc = pl.pallas_call(
        functools.partial(_body, s_blk=s_blk, s=s, v=v, nb=b, nj=nj),
        grid=(b, nj),
        in_specs=[
            pl.BlockSpec(memory_space=pltpu.SMEM),
            pl.BlockSpec(memory_space=pltpu.SMEM),
            pl.BlockSpec((1, s_blk, v), lambda bi, ji: (bi, ji, 0)),
            pl.BlockSpec((1, v), lambda bi, ji: (0, 0)),
            pl.BlockSpec((1, 1, s_blk), lambda bi, ji: (bi * nj + ji, 0, 0)),
            pl.BlockSpec((1, 1, s_blk), lambda bi, ji: (bi * nj + ji, 0, 0)),
        ],
        out_specs=[
            pl.BlockSpec((1, 1, s_blk), lambda bi, ji: (bi * nj + ji, 0, 0)),
            pl.BlockSpec((b, s), lambda bi, ji: (0, 0)),
        ],
        out_shape=[
            jax.ShapeDtypeStruct((b * nj, 1, s_blk), jnp.int32),
            jax.ShapeDtypeStruct((b, s), jnp.int32),
        ],
        scratch_shapes=[pltpu.VMEM((1, b * s), jnp.float32)],
    )(temperature.reshape(1), mask_len.reshape(1), logits, vidx, gumbel3, mask3)

    return zpred.reshape(b, s), maskbc.astype(jnp.bool_)


# speedup vs baseline: 1.3001x; 1.0732x over previous
import functools

import jax
import jax.numpy as jnp
from jax.experimental import pallas as pl
from jax.experimental.pallas import tpu as pltpu

_CHOICE_TEMPERATURE = 4.5
_INT_MIN = -2147483648


def _body(temp_ref, ml_ref, logits_ref, vidx_ref, gumbel_ref, mask_ref,
          zpred_ref, maskbc_ref, conf_ref, *, s_blk, s, v, nb, nj):
    bi = pl.program_id(0)
    j = pl.program_id(1)
    x = logits_ref[0]
    m = jnp.max(x, axis=-1)
    cand = jnp.where(x == m[:, None], vidx_ref[:], v)
    amax = jnp.min(cand, axis=-1)
    se = jnp.sum(jnp.exp(x - m[:, None]), axis=-1)
    pmax = 1.0 / se
    temp = temp_ref[0]
    g = gumbel_ref[0, 0]
    mk = mask_ref[0, 0]
    conf = jnp.where(mk != 0, pmax + temp * g, jnp.inf)
    zpred_ref[0, 0, :] = amax
    conf_ref[0, pl.ds(bi * s + j * s_blk, s_blk)] = conf

    @pl.when((bi == nb - 1) & (j == nj - 1))
    def _rank():
        c = conf_ref[0, :].reshape(nb, s)
        cb = jax.lax.bitcast_convert_type(c, jnp.int32)
        key = jnp.where(cb < 0, cb ^ 0x7FFFFFFF, cb)
        k = ml_ref[0]
        imin = jnp.int32(_INT_MIN)

        def bit1(i, res_u):
            cand_u = res_u | (jnp.int32(1) << (31 - i))
            cnt = jnp.sum((key < (cand_u ^ imin)).astype(jnp.int32),
                          axis=1, keepdims=True)
            return jnp.where(cnt < k, cand_u, res_u)

        res_u = jax.lax.fori_loop(0, 32, bit1,
                                  jnp.zeros((nb, 1), jnp.int32))
        t_s = res_u ^ imin
        lt = key < t_s
        eq = key == t_s
        jrem = k - jnp.sum(lt.astype(jnp.int32), axis=1, keepdims=True)
        idx = vidx_ref[:, :s]

        def bit2(i, res2):
            cand2 = res2 | (jnp.int32(1) << (9 - i))
            cnt = jnp.sum((eq & (idx < cand2)).astype(jnp.int32),
                          axis=1, keepdims=True)
            return jnp.where(cnt < jrem, cand2, res2)

        t_idx = jax.lax.fori_loop(0, 10, bit2,
                                  jnp.zeros((nb, 1), jnp.int32))
        maskbc_ref[:, :] = (lt | (eq & (idx <= t_idx))).astype(jnp.int32)


def kernel(logits, ratio, gumbel, z_indices, mask, mask_num):
    del z_indices
    b, s, v = logits.shape
    s_blk = 512
    nj = s // s_blk

    r = ratio[0]
    mask_ratio = jnp.cos(r * jnp.pi / 2.0)
    mask_len = jnp.maximum(jnp.ceil(mask_num * mask_ratio), 1.0).astype(jnp.int32)
    temperature = (_CHOICE_TEMPERATURE * (1.0 - mask_ratio)).astype(jnp.float32)

    vidx = jnp.arange(v, dtype=jnp.int32).reshape(1, v)
    gumbel3 = gumbel.reshape(b * nj, 1, s_blk)
    mask3 = mask.astype(jnp.int32).reshape(b * nj, 1, s_blk)

    zpred, maskbc = pl.pallas_call(
        functools.partial(_body, s_blk=s_blk, s=s, v=v, nb=b, nj=nj),
        grid=(b, nj),
        in_specs=[
            pl.BlockSpec(memory_space=pltpu.SMEM),
            pl.BlockSpec(memory_space=pltpu.SMEM),
            pl.BlockSpec((1, s_blk, v), lambda bi, ji: (bi, ji, 0)),
            pl.BlockSpec((1, v), lambda bi, ji: (0, 0)),
            pl.BlockSpec((1, 1, s_blk), lambda bi, ji: (bi * nj + ji, 0, 0)),
            pl.BlockSpec((1, 1, s_blk), lambda bi, ji: (bi * nj + ji, 0, 0)),
        ],
        out_specs=[
            pl.BlockSpec((1, 1, s_blk), lambda bi, ji: (bi * nj + ji, 0, 0)),
            pl.BlockSpec((b, s), lambda bi, ji: (0, 0)),
        ],
        out_shape=[
            jax.ShapeDtypeStruct((b * nj, 1, s_blk), jnp.int32),
            jax.ShapeDtypeStruct((b, s), jnp.int32),
        ],
        scratch_shapes=[pltpu.VMEM((1, b * s), jnp.float32)],
    )(temperature.reshape(1), mask_len.reshape(1), logits, vidx, gumbel3, mask3)

    return zpred.reshape(b, s), maskbc.astype(jnp.bool_)
